# Initial kernel scaffold; baseline (speedup 1.0000x reference)
import jax
import jax.numpy as jnp
from jax.experimental import pallas as pl


def kernel(x, W1, b1, W2, b2, Wg, bg):
    raise NotImplementedError("write your pallas kernel here")


# speedup vs baseline: 1.2611x; 1.2611x over previous
import jax
import jax.numpy as jnp
from jax.experimental import pallas as pl
from jax.experimental.pallas import tpu as pltpu

D_MODEL = 1024
D_FF = 2048
NUM_EXPERTS = 8
N_TOKENS = 2048
BT = 256


def _moe_kernel(x_ref, W1_ref, b1_ref, W2_ref, b2_ref, Wg_ref, bg_ref,
                out_ref):
    e = pl.program_id(0)
    t = pl.program_id(1)
    x = x_ref[...]

    logits = jnp.dot(x.astype(jnp.bfloat16), Wg_ref[...].astype(jnp.bfloat16),
                     preferred_element_type=jnp.float32) + bg_ref[0]
    ii = jax.lax.broadcasted_iota(jnp.int32, logits.shape, 1)
    m1 = jnp.max(logits, axis=-1, keepdims=True)
    i1 = jnp.min(jnp.where(logits == m1, ii, NUM_EXPERTS),
                 axis=-1, keepdims=True)
    logits2 = jnp.where(ii == i1, -jnp.inf, logits)
    m2 = jnp.max(logits2, axis=-1, keepdims=True)
    i2 = jnp.min(jnp.where(logits2 == m2, ii, NUM_EXPERTS),
                 axis=-1, keepdims=True)
    g1 = jax.nn.sigmoid(m1 - m2)
    g2 = 1.0 - g1
    w_e = jnp.where(i1 == e, g1, 0.0) + jnp.where(i2 == e, g2, 0.0)

    xb = x.astype(jnp.bfloat16)
    w1 = W1_ref[0].astype(jnp.bfloat16)
    h = jnp.dot(xb, w1, preferred_element_type=jnp.float32) + b1_ref[0, 0]
    hb = jnp.maximum(h, 0.0).astype(jnp.bfloat16)
    w2 = W2_ref[0].astype(jnp.bfloat16)
    o = jnp.dot(hb, w2, preferred_element_type=jnp.float32) + b2_ref[0, 0]
    contrib = o * w_e

    sl = pl.ds(t * BT, BT)

    @pl.when(e == 0)
    def _():
        out_ref[sl, :] = contrib

    @pl.when(e != 0)
    def _():
        out_ref[sl, :] = out_ref[sl, :] + contrib


def kernel(x, W1, b1, W2, b2, Wg, bg):
    bg2 = bg.reshape(1, NUM_EXPERTS)
    b1r = b1.reshape(NUM_EXPERTS, 1, D_FF)
    b2r = b2.reshape(NUM_EXPERTS, 1, D_MODEL)
    grid = (NUM_EXPERTS, N_TOKENS // BT)
    return pl.pallas_call(
        _moe_kernel,
        grid=grid,
        in_specs=[
            pl.BlockSpec((BT, D_MODEL), lambda e, t: (t, 0)),
            pl.BlockSpec((1, D_MODEL, D_FF), lambda e, t: (e, 0, 0)),
            pl.BlockSpec((1, 1, D_FF), lambda e, t: (e, 0, 0)),
            pl.BlockSpec((1, D_FF, D_MODEL), lambda e, t: (e, 0, 0)),
            pl.BlockSpec((1, 1, D_MODEL), lambda e, t: (e, 0, 0)),
            pl.BlockSpec((D_MODEL, NUM_EXPERTS), lambda e, t: (0, 0)),
            pl.BlockSpec((1, NUM_EXPERTS), lambda e, t: (0, 0)),
        ],
        out_specs=pl.BlockSpec((N_TOKENS, D_MODEL), lambda e, t: (0, 0)),
        out_shape=jax.ShapeDtypeStruct((N_TOKENS, D_MODEL), jnp.float32),
        compiler_params=pltpu.CompilerParams(
            dimension_semantics=("arbitrary", "arbitrary"),
        ),
    )(x, W1, b1r, W2, b2r, Wg, bg2)
